# baseline (speedup 1.0000x reference)
import functools

import jax
import jax.numpy as jnp
from jax import lax
from jax.experimental import pallas as pl
from jax.experimental.pallas import tpu as pltpu
from jax.experimental.pallas import tpu_sc as plsc

ITEM_COUNT = 100000
EMB_HALF = 16
B = 16384
L = 200
HID = 64

NC, NS = 2, 16
NW = NC * NS

ROWS_PER_W_A = 3200
CHUNK_A = 800
FB_ROWS = NW * ROWS_PER_W_A

B_PER_W = B // NW
NB = 4
CHUNK = NB * L
GROUP_B = 16
GROUP_IDX = GROUP_B * L
SUBS = GROUP_B // NB
NGROUPS = B_PER_W // GROUP_B
OUTW = 80

_mesh = plsc.VectorSubcoreMesh(core_axis_name="c", subcore_axis_name="s",
                               num_cores=NC, num_subcores=NS)
_sc_params = pltpu.CompilerParams(use_tc_tiling_on_sc=False,
                                  needs_layout_passes=False)
_PK = plsc.PackFormat.INTERLEAVED


@functools.partial(
    pl.kernel,
    out_type=jax.ShapeDtypeStruct((FB_ROWS, 2 * EMB_HALF), jnp.bfloat16),
    mesh=_mesh,
    scratch_types=[
        pltpu.VMEM((2, CHUNK_A), jnp.int32),
        pltpu.VMEM((2, CHUNK_A, EMB_HALF), jnp.float32),
        pltpu.VMEM((2, CHUNK_A, EMB_HALF), jnp.float32),
        pltpu.VMEM((2, CHUNK_A, 2 * EMB_HALF), jnp.bfloat16),
        pltpu.SemaphoreType.DMA,
        pltpu.SemaphoreType.DMA,
        pltpu.SemaphoreType.DMA,
        pltpu.SemaphoreType.DMA,
        pltpu.SemaphoreType.DMA,
        pltpu.SemaphoreType.DMA,
    ],
    compiler_params=_sc_params,
)
def _build_fb(item_emb_w, cate_emb_w, cate_list, fb_out,
              cidx_v, irows_v, crows_v, fused_v,
              sem_i0, sem_i1, sem_g0, sem_g1, sem_o0, sem_o1):
    wid = lax.axis_index("s") * NC + lax.axis_index("c")
    base = jnp.where(wid == NW - 1, ITEM_COUNT - ROWS_PER_W_A,
                     wid * ROWS_PER_W_A)
    sem_i = (sem_i0, sem_i1)
    sem_g = (sem_g0, sem_g1)
    sem_o = (sem_o0, sem_o1)
    NCH = ROWS_PER_W_A // CHUNK_A

    def in_copy(c, par):
        bc = pl.multiple_of(base + c * CHUNK_A, 8)
        pltpu.async_copy(cate_list.at[pl.ds(bc, CHUNK_A)], cidx_v.at[par],
                         sem_i[par])
        pltpu.async_copy(item_emb_w.at[pl.ds(bc, CHUNK_A), :],
                         irows_v.at[par], sem_i[par])

    def in_wait(par):
        pltpu.make_async_copy(cate_list.at[pl.ds(0, CHUNK_A)],
                              cidx_v.at[par], sem_i[par]).wait()
        pltpu.make_async_copy(item_emb_w.at[pl.ds(0, CHUNK_A), :],
                              irows_v.at[par], sem_i[par]).wait()

    def gather(par):
        pltpu.async_copy(cate_emb_w.at[cidx_v.at[par]], crows_v.at[par],
                         sem_g[par])

    def gather_wait(par):
        pltpu.make_async_copy(cate_emb_w.at[cidx_v.at[0]], crows_v.at[par],
                              sem_g[par]).wait()

    def pack_chunk(par):
        def prow(i, _):
            for u in range(8):
                r = i * 8 + u
                fused_v[par, r, :] = plsc.pack(irows_v[par, r, :],
                                               crows_v[par, r, :], format=_PK)
            return 0
        lax.fori_loop(0, CHUNK_A // 8, prow, 0)

    def out_copy(c, par):
        bc = pl.multiple_of(base + c * CHUNK_A, 8)
        pltpu.async_copy(fused_v.at[par], fb_out.at[pl.ds(bc, CHUNK_A)],
                         sem_o[par])

    def out_wait(par):
        pltpu.make_async_copy(fused_v.at[par], fb_out.at[pl.ds(0, CHUNK_A)],
                              sem_o[par]).wait()

    in_copy(0, 0)
    in_copy(1, 1)
    in_wait(0)
    gather(0)
    for c in range(NCH):
        par = c % 2
        if c + 1 < NCH:
            in_wait(1 - par)
            gather(1 - par)
        gather_wait(par)
        if c >= 2:
            out_wait(par)
        pack_chunk(par)
        out_copy(c, par)
        if c + 2 < NCH:
            in_copy(c + 2, par)
    for par in range(min(2, NCH)):
        out_wait(par)


@functools.partial(
    pl.kernel,
    out_type=jax.ShapeDtypeStruct((B, OUTW), jnp.float32),
    mesh=_mesh,
    scratch_types=[
        pltpu.VMEM((2, GROUP_B, L), jnp.int32),
        pltpu.VMEM((4, CHUNK, 2 * EMB_HALF), jnp.bfloat16),
        pltpu.VMEM((B_PER_W, OUTW), jnp.float32),
        pltpu.VMEM((B_PER_W,), jnp.int32),
        pltpu.VMEM((B_PER_W, 2 * EMB_HALF), jnp.bfloat16),
        pltpu.VMEM((B_PER_W,), jnp.float32),
        pltpu.SemaphoreType.DMA,
        pltpu.SemaphoreType.DMA,
        pltpu.SemaphoreType.DMA,
        pltpu.SemaphoreType.DMA,
        pltpu.SemaphoreType.DMA,
        pltpu.SemaphoreType.DMA,
        pltpu.SemaphoreType.DMA,
    ],
    compiler_params=_sc_params,
)
def _hist_sums(fb, hist2d, item, bias_flat,
               us_all_hbm,
               idx_v, rows, us_st,
               item_idx_v, ie_rows_v, bias_v,
               sem_idx0, sem_idx1, sem_rows0, sem_rows1, sem_rows2,
               sem_rows3, sem_misc):
    wid = lax.axis_index("s") * NC + lax.axis_index("c")
    base_b = wid * B_PER_W

    pltpu.sync_copy(item.at[pl.ds(pl.multiple_of(base_b, 8), B_PER_W)],
                    item_idx_v)
    pltpu.async_copy(fb.at[item_idx_v], ie_rows_v, sem_misc)
    pltpu.async_copy(bias_flat.at[item_idx_v], bias_v, sem_misc)

    sem_idx = (sem_idx0, sem_idx1)
    sem_rows = (sem_rows0, sem_rows1, sem_rows2, sem_rows3)

    def idx_copy(g, par):
        gb = pl.multiple_of(base_b + g * GROUP_B, 8)
        pltpu.async_copy(hist2d.at[pl.ds(gb, GROUP_B), :], idx_v.at[par],
                         sem_idx[par])

    def idx_wait(par):
        pltpu.make_async_copy(hist2d.at[pl.ds(0, GROUP_B), :],
                              idx_v.at[par], sem_idx[par]).wait()

    def issue_gather(ipar, off, rp):
        for j in range(NB):
            sl = idx_v.at[ipar, off // L + j, :]
            pltpu.async_copy(fb.at[sl], rows.at[rp, pl.ds(j * L, L), :],
                             sem_rows[rp])

    def rows_wait(rp):
        for j in range(NB):
            pltpu.make_async_copy(fb.at[idx_v.at[0, 0, :]],
                                  rows.at[rp, pl.ds(j * L, L), :],
                                  sem_rows[rp]).wait()

    UNROLL = 40
    NSTEPS = L // UNROLL

    def acc_chunk(c, rp):
        def bbody(b4, _):
            zf = jnp.zeros((EMB_HALF,), jnp.float32)

            def lbody(m, accs):
                ai, ac = accs
                r0 = b4 * L + m * UNROLL
                for h in range(2):
                    zb = [jnp.zeros((2 * EMB_HALF,), jnp.bfloat16)
                          for _ in range(4)]
                    for l in range(UNROLL // 2):
                        r = r0 + h * (UNROLL // 2) + l
                        zb[l % 4] = zb[l % 4] + rows[rp, r, :]
                    zz = (zb[0] + zb[1]) + (zb[2] + zb[3])
                    pi, pc = plsc.unpack(zz, format=_PK)
                    ai = ai + pi
                    ac = ac + pc
                return ai, ac

            ui, uc = lax.fori_loop(0, NSTEPS, lbody, (zf, zf))
            b_local = c * NB + b4
            us_st[b_local, pl.ds(0, EMB_HALF)] = ui
            us_st[b_local, pl.ds(EMB_HALF, EMB_HALF)] = uc
            return 0

        lax.fori_loop(0, NB, bbody, 0)

    def group_body(g, gpar):
        for s in range(SUBS):
            c = g * SUBS + s
            rp = s
            if s < SUBS - 2:
                issue_gather(gpar, (s + 2) * CHUNK, s + 2)
            elif s == SUBS - 2:
                @pl.when(g + 1 < NGROUPS)
                def _():
                    idx_wait(1 - gpar)
                    issue_gather(1 - gpar, 0, 0)
            else:
                @pl.when(g + 1 < NGROUPS)
                def _():
                    issue_gather(1 - gpar, CHUNK, 1)
            rows_wait(rp)
            acc_chunk(c, rp)
        @pl.when(g + 2 < NGROUPS)
        def _():
            idx_copy(g + 2, gpar)

    idx_copy(0, 0)
    idx_copy(1, 1)
    idx_wait(0)
    issue_gather(0, 0, 0)
    issue_gather(0, CHUNK, 1)

    def macro(i, _):
        group_body(2 * i, 0)
        group_body(2 * i + 1, 1)
        return 0

    lax.fori_loop(0, NGROUPS // 2, macro, 0)

    pltpu.make_async_copy(fb.at[item_idx_v], ie_rows_v, sem_misc).wait()
    pltpu.make_async_copy(bias_flat.at[item_idx_v], bias_v, sem_misc).wait()

    def ie_row(i, _):
        for u in range(8):
            r = i * 8 + u
            pi, pc = plsc.unpack(ie_rows_v[r, :], format=_PK)
            us_st[r, pl.ds(2 * EMB_HALF, EMB_HALF)] = pi
            us_st[r, pl.ds(3 * EMB_HALF, EMB_HALF)] = pc
        return 0

    lax.fori_loop(0, B_PER_W // 8, ie_row, 0)

    lane = lax.iota(jnp.int32, 16)
    col = jnp.full((16,), 4 * EMB_HALF, jnp.int32)

    def bias_row(i, _):
        vals = bias_v[pl.ds(i * 16, 16)]
        plsc.store_scatter(us_st, [lane + i * 16, col], vals)
        return 0

    lax.fori_loop(0, B_PER_W // 16, bias_row, 0)

    ob = pl.ds(pl.multiple_of(base_b, 8), B_PER_W)
    pltpu.sync_copy(us_st, us_all_hbm.at[ob, :])


def _tail_body(xa, w1, b1r, w2, b2r, o):
    inv_l = 1.0 / L
    x = xa[...]
    ue = x[:, 0:2 * EMB_HALF] * inv_l
    ie = x[:, 2 * EMB_HALF:4 * EMB_HALF]
    bz = x[:, 4 * EMB_HALF:4 * EMB_HALF + 1]
    wide = jnp.sum(ue * ie, axis=1, keepdims=True)
    xx = jnp.concatenate([ue, ie], axis=1)
    h = jnp.maximum(
        jnp.dot(xx, w1[...], preferred_element_type=jnp.float32) + b1r[...],
        0.0)
    deep = jnp.dot(h, w2[...], preferred_element_type=jnp.float32) + b2r[...]
    o[...] = jax.nn.sigmoid(bz + wide + deep)


def _tail(us_all, W1, b1r, W2, b2r):
    BLK = 2048
    grid = (B // BLK,)
    return pl.pallas_call(
        _tail_body,
        grid=grid,
        in_specs=[pl.BlockSpec((BLK, OUTW), lambda i: (i, 0)),
                  pl.BlockSpec((2 * 2 * EMB_HALF, HID), lambda i: (0, 0)),
                  pl.BlockSpec((1, HID), lambda i: (0, 0)),
                  pl.BlockSpec((HID, 1), lambda i: (0, 0)),
                  pl.BlockSpec((1, 1), lambda i: (0, 0))],
        out_specs=pl.BlockSpec((BLK, 1), lambda i: (i, 0)),
        out_shape=jax.ShapeDtypeStruct((B, 1), jnp.float32),
    )(us_all, W1, b1r, W2, b2r)


def kernel(item_emb_w, cate_emb_w, item_bias, W1, b1, W2, b2,
           cate_list, user, item, hist_item, neg_hist_item):
    fb = _build_fb(item_emb_w, cate_emb_w, cate_list)
    us_all = _hist_sums(fb, hist_item, item, item_bias.reshape(-1))
    return _tail(us_all, W1, b1.reshape(1, HID), W2, b2.reshape(1, 1))


# speedup vs baseline: 1.0067x; 1.0067x over previous
import functools

import jax
import jax.numpy as jnp
from jax import lax
from jax.experimental import pallas as pl
from jax.experimental.pallas import tpu as pltpu
from jax.experimental.pallas import tpu_sc as plsc

ITEM_COUNT = 100000
EMB_HALF = 16
B = 16384
L = 200
HID = 64

NC, NS = 2, 16
NW = NC * NS

ROWS_PER_W_A = 3200
CHUNK_A = 800
FB_ROWS = NW * ROWS_PER_W_A

B_PER_W = B // NW
NB = 4
CHUNK = NB * L
GROUP_B = 16
GROUP_IDX = GROUP_B * L
SUBS = GROUP_B // NB
NGROUPS = B_PER_W // GROUP_B
OUTW = 80

_mesh = plsc.VectorSubcoreMesh(core_axis_name="c", subcore_axis_name="s",
                               num_cores=NC, num_subcores=NS)
_sc_params = pltpu.CompilerParams(use_tc_tiling_on_sc=False,
                                  needs_layout_passes=False)
_PK = plsc.PackFormat.INTERLEAVED


@functools.partial(
    pl.kernel,
    out_type=jax.ShapeDtypeStruct((FB_ROWS, 2 * EMB_HALF), jnp.bfloat16),
    mesh=_mesh,
    scratch_types=[
        pltpu.VMEM((2, CHUNK_A), jnp.int32),
        pltpu.VMEM((2, CHUNK_A, EMB_HALF), jnp.float32),
        pltpu.VMEM((2, CHUNK_A, EMB_HALF), jnp.float32),
        pltpu.VMEM((2, CHUNK_A, 2 * EMB_HALF), jnp.bfloat16),
        pltpu.SemaphoreType.DMA,
        pltpu.SemaphoreType.DMA,
        pltpu.SemaphoreType.DMA,
        pltpu.SemaphoreType.DMA,
        pltpu.SemaphoreType.DMA,
        pltpu.SemaphoreType.DMA,
    ],
    compiler_params=_sc_params,
)
def _build_fb(item_emb_w, cate_emb_w, cate_list, fb_out,
              cidx_v, irows_v, crows_v, fused_v,
              sem_i0, sem_i1, sem_g0, sem_g1, sem_o0, sem_o1):
    wid = lax.axis_index("s") * NC + lax.axis_index("c")
    base = jnp.where(wid == NW - 1, ITEM_COUNT - ROWS_PER_W_A,
                     wid * ROWS_PER_W_A)
    sem_i = (sem_i0, sem_i1)
    sem_g = (sem_g0, sem_g1)
    sem_o = (sem_o0, sem_o1)
    NCH = ROWS_PER_W_A // CHUNK_A

    def in_copy(c, par):
        bc = pl.multiple_of(base + c * CHUNK_A, 8)
        pltpu.async_copy(cate_list.at[pl.ds(bc, CHUNK_A)], cidx_v.at[par],
                         sem_i[par])
        pltpu.async_copy(item_emb_w.at[pl.ds(bc, CHUNK_A), :],
                         irows_v.at[par], sem_i[par])

    def in_wait(par):
        pltpu.make_async_copy(cate_list.at[pl.ds(0, CHUNK_A)],
                              cidx_v.at[par], sem_i[par]).wait()
        pltpu.make_async_copy(item_emb_w.at[pl.ds(0, CHUNK_A), :],
                              irows_v.at[par], sem_i[par]).wait()

    def gather(par):
        pltpu.async_copy(cate_emb_w.at[cidx_v.at[par]], crows_v.at[par],
                         sem_g[par])

    def gather_wait(par):
        pltpu.make_async_copy(cate_emb_w.at[cidx_v.at[0]], crows_v.at[par],
                              sem_g[par]).wait()

    def pack_chunk(par):
        def prow(i, _):
            for u in range(8):
                r = i * 8 + u
                fused_v[par, r, :] = plsc.pack(irows_v[par, r, :],
                                               crows_v[par, r, :], format=_PK)
            return 0
        lax.fori_loop(0, CHUNK_A // 8, prow, 0)

    def out_copy(c, par):
        bc = pl.multiple_of(base + c * CHUNK_A, 8)
        pltpu.async_copy(fused_v.at[par], fb_out.at[pl.ds(bc, CHUNK_A)],
                         sem_o[par])

    def out_wait(par):
        pltpu.make_async_copy(fused_v.at[par], fb_out.at[pl.ds(0, CHUNK_A)],
                              sem_o[par]).wait()

    in_copy(0, 0)
    in_copy(1, 1)
    in_wait(0)
    gather(0)
    for c in range(NCH):
        par = c % 2
        if c + 1 < NCH:
            in_wait(1 - par)
            gather(1 - par)
        gather_wait(par)
        if c >= 2:
            out_wait(par)
        pack_chunk(par)
        out_copy(c, par)
        if c + 2 < NCH:
            in_copy(c + 2, par)
    for par in range(min(2, NCH)):
        out_wait(par)


@functools.partial(
    pl.kernel,
    out_type=jax.ShapeDtypeStruct((B, OUTW), jnp.float32),
    mesh=_mesh,
    scratch_types=[
        pltpu.VMEM((2, GROUP_B, L), jnp.int32),
        pltpu.VMEM((4, CHUNK, 2 * EMB_HALF), jnp.bfloat16),
        pltpu.VMEM((B_PER_W, OUTW), jnp.float32),
        pltpu.VMEM((B_PER_W,), jnp.int32),
        pltpu.VMEM((B_PER_W, 2 * EMB_HALF), jnp.bfloat16),
        pltpu.VMEM((B_PER_W,), jnp.float32),
        pltpu.SemaphoreType.DMA,
        pltpu.SemaphoreType.DMA,
        pltpu.SemaphoreType.DMA,
        pltpu.SemaphoreType.DMA,
        pltpu.SemaphoreType.DMA,
        pltpu.SemaphoreType.DMA,
        pltpu.SemaphoreType.DMA,
    ],
    compiler_params=_sc_params,
)
def _hist_sums(fb, hist2d, item, bias_flat,
               us_all_hbm,
               idx_v, rows, us_st,
               item_idx_v, ie_rows_v, bias_v,
               sem_idx0, sem_idx1, sem_rows0, sem_rows1, sem_rows2,
               sem_rows3, sem_misc):
    wid = lax.axis_index("s") * NC + lax.axis_index("c")
    base_b = wid * B_PER_W

    pltpu.sync_copy(item.at[pl.ds(pl.multiple_of(base_b, 8), B_PER_W)],
                    item_idx_v)
    pltpu.async_copy(fb.at[item_idx_v], ie_rows_v, sem_misc)
    pltpu.async_copy(bias_flat.at[item_idx_v], bias_v, sem_misc)

    sem_idx = (sem_idx0, sem_idx1)
    sem_rows = (sem_rows0, sem_rows1, sem_rows2, sem_rows3)

    def idx_copy(g, par):
        gb = pl.multiple_of(base_b + g * GROUP_B, 8)
        pltpu.async_copy(hist2d.at[pl.ds(gb, GROUP_B), :], idx_v.at[par],
                         sem_idx[par])

    def idx_wait(par):
        pltpu.make_async_copy(hist2d.at[pl.ds(0, GROUP_B), :],
                              idx_v.at[par], sem_idx[par]).wait()

    def issue_gather(ipar, off, rp):
        for j in range(NB):
            sl = idx_v.at[ipar, off // L + j, :]
            pltpu.async_copy(fb.at[sl], rows.at[rp, pl.ds(j * L, L), :],
                             sem_rows[rp])

    def rows_wait(rp):
        for j in range(NB):
            pltpu.make_async_copy(fb.at[idx_v.at[0, 0, :]],
                                  rows.at[rp, pl.ds(j * L, L), :],
                                  sem_rows[rp]).wait()

    UNROLL = 40
    NSTEPS = L // UNROLL

    def acc_chunk(c, rp):
        def bbody(b4, _):
            zf = jnp.zeros((EMB_HALF,), jnp.float32)

            def lbody(m, accs):
                ai, ac = accs
                r0 = b4 * L + m * UNROLL
                for h in range(2):
                    zb = [jnp.zeros((2 * EMB_HALF,), jnp.bfloat16)
                          for _ in range(4)]
                    for l in range(UNROLL // 2):
                        r = r0 + h * (UNROLL // 2) + l
                        zb[l % 4] = zb[l % 4] + rows[rp, r, :]
                    zz = (zb[0] + zb[1]) + (zb[2] + zb[3])
                    pi, pc = plsc.unpack(zz, format=_PK)
                    ai = ai + pi
                    ac = ac + pc
                return ai, ac

            ui, uc = lax.fori_loop(0, NSTEPS, lbody, (zf, zf))
            b_local = c * NB + b4
            us_st[b_local, pl.ds(0, EMB_HALF)] = ui
            us_st[b_local, pl.ds(EMB_HALF, EMB_HALF)] = uc
            return 0

        lax.fori_loop(0, NB, bbody, 0)

    def group_body(g, gpar):
        for s in range(SUBS):
            c = g * SUBS + s
            rp = s
            if s < SUBS - 2:
                issue_gather(gpar, (s + 2) * CHUNK, s + 2)
            elif s == SUBS - 2:
                @pl.when(g + 1 < NGROUPS)
                def _():
                    idx_wait(1 - gpar)
                    issue_gather(1 - gpar, 0, 0)
            else:
                @pl.when(g + 1 < NGROUPS)
                def _():
                    issue_gather(1 - gpar, CHUNK, 1)
            rows_wait(rp)
            acc_chunk(c, rp)
        @pl.when(g + 2 < NGROUPS)
        def _():
            idx_copy(g + 2, gpar)

    idx_copy(0, 0)
    idx_copy(1, 1)
    idx_wait(0)
    issue_gather(0, 0, 0)
    issue_gather(0, CHUNK, 1)

    def macro(i, _):
        group_body(2 * i, 0)
        group_body(2 * i + 1, 1)
        return 0

    lax.fori_loop(0, NGROUPS // 2, macro, 0)

    pltpu.make_async_copy(fb.at[item_idx_v], ie_rows_v, sem_misc).wait()
    pltpu.make_async_copy(bias_flat.at[item_idx_v], bias_v, sem_misc).wait()

    def ie_row(i, _):
        for u in range(8):
            r = i * 8 + u
            pi, pc = plsc.unpack(ie_rows_v[r, :], format=_PK)
            us_st[r, pl.ds(2 * EMB_HALF, EMB_HALF)] = pi
            us_st[r, pl.ds(3 * EMB_HALF, EMB_HALF)] = pc
        return 0

    lax.fori_loop(0, B_PER_W // 8, ie_row, 0)

    lane = lax.iota(jnp.int32, 16)
    col = jnp.full((16,), 4 * EMB_HALF, jnp.int32)

    def bias_row(i, _):
        vals = bias_v[pl.ds(i * 16, 16)]
        plsc.store_scatter(us_st, [lane + i * 16, col], vals)
        return 0

    lax.fori_loop(0, B_PER_W // 16, bias_row, 0)

    ob = pl.ds(pl.multiple_of(base_b, 8), B_PER_W)
    pltpu.sync_copy(us_st, us_all_hbm.at[ob, :])


def _tail_body(xa, w1, b1r, w2, b2r, o):
    inv_l = 1.0 / L
    x = xa[...]
    ue = x[:, 0:2 * EMB_HALF] * inv_l
    ie = x[:, 2 * EMB_HALF:4 * EMB_HALF]
    bz = x[:, 4 * EMB_HALF:4 * EMB_HALF + 1]
    wide = jnp.sum(ue * ie, axis=1, keepdims=True)
    xx = jnp.concatenate([ue, ie], axis=1)
    h = jnp.maximum(
        jnp.dot(xx, w1[...], preferred_element_type=jnp.float32) + b1r[...],
        0.0)
    deep = jnp.dot(h, w2[...], preferred_element_type=jnp.float32) + b2r[...]
    o[...] = jax.nn.sigmoid(bz + wide + deep)[:, 0]


def _tail(us_all, W1, b1r, W2, b2r):
    return pl.pallas_call(
        _tail_body,
        grid=(1,),
        in_specs=[pl.BlockSpec((B, OUTW), lambda i: (0, 0)),
                  pl.BlockSpec((2 * 2 * EMB_HALF, HID), lambda i: (0, 0)),
                  pl.BlockSpec((1, HID), lambda i: (0, 0)),
                  pl.BlockSpec((HID, 1), lambda i: (0, 0)),
                  pl.BlockSpec((1, 1), lambda i: (0, 0))],
        out_specs=pl.BlockSpec((B,), lambda i: (0,)),
        out_shape=jax.ShapeDtypeStruct((B,), jnp.float32),
    )(us_all, W1, b1r, W2, b2r)


def kernel(item_emb_w, cate_emb_w, item_bias, W1, b1, W2, b2,
           cate_list, user, item, hist_item, neg_hist_item):
    fb = _build_fb(item_emb_w, cate_emb_w, cate_list)
    us_all = _hist_sums(fb, hist_item, item, item_bias.reshape(-1))
    out = _tail(us_all, W1, b1.reshape(1, HID), W2, b2.reshape(1, 1))
    return out.reshape(B, 1)


# speedup vs baseline: 1.0114x; 1.0046x over previous
import functools

import jax
import jax.numpy as jnp
from jax import lax
from jax.experimental import pallas as pl
from jax.experimental.pallas import tpu as pltpu
from jax.experimental.pallas import tpu_sc as plsc

ITEM_COUNT = 100000
EMB_HALF = 16
B = 16384
L = 200
HID = 64

NC, NS = 2, 16
NW = NC * NS

ROWS_PER_W_A = 3200
CHUNK_A = 800
FB_ROWS = NW * ROWS_PER_W_A

B_PER_W = B // NW
NB = 4
CHUNK = NB * L
GROUP_B = 16
GROUP_IDX = GROUP_B * L
SUBS = GROUP_B // NB
NGROUPS = B_PER_W // GROUP_B
OUTW = 80

_mesh = plsc.VectorSubcoreMesh(core_axis_name="c", subcore_axis_name="s",
                               num_cores=NC, num_subcores=NS)
_sc_params = pltpu.CompilerParams(use_tc_tiling_on_sc=False,
                                  needs_layout_passes=False)
_PK = plsc.PackFormat.INTERLEAVED


@functools.partial(
    pl.kernel,
    out_type=jax.ShapeDtypeStruct((FB_ROWS, 2 * EMB_HALF), jnp.bfloat16),
    mesh=_mesh,
    scratch_types=[
        pltpu.VMEM((2, CHUNK_A), jnp.int32),
        pltpu.VMEM((2, CHUNK_A, EMB_HALF), jnp.float32),
        pltpu.VMEM((2, CHUNK_A, EMB_HALF), jnp.float32),
        pltpu.VMEM((2, CHUNK_A, 2 * EMB_HALF), jnp.bfloat16),
        pltpu.SemaphoreType.DMA,
        pltpu.SemaphoreType.DMA,
        pltpu.SemaphoreType.DMA,
        pltpu.SemaphoreType.DMA,
        pltpu.SemaphoreType.DMA,
        pltpu.SemaphoreType.DMA,
    ],
    compiler_params=_sc_params,
)
def _build_fb(item_emb_w, cate_emb_w, cate_list, fb_out,
              cidx_v, irows_v, crows_v, fused_v,
              sem_i0, sem_i1, sem_g0, sem_g1, sem_o0, sem_o1):
    wid = lax.axis_index("s") * NC + lax.axis_index("c")
    base = jnp.where(wid == NW - 1, ITEM_COUNT - ROWS_PER_W_A,
                     wid * ROWS_PER_W_A)
    sem_i = (sem_i0, sem_i1)
    sem_g = (sem_g0, sem_g1)
    sem_o = (sem_o0, sem_o1)
    NCH = ROWS_PER_W_A // CHUNK_A

    def in_copy(c, par):
        bc = pl.multiple_of(base + c * CHUNK_A, 8)
        pltpu.async_copy(cate_list.at[pl.ds(bc, CHUNK_A)], cidx_v.at[par],
                         sem_i[par])
        pltpu.async_copy(item_emb_w.at[pl.ds(bc, CHUNK_A), :],
                         irows_v.at[par], sem_i[par])

    def in_wait(par):
        pltpu.make_async_copy(cate_list.at[pl.ds(0, CHUNK_A)],
                              cidx_v.at[par], sem_i[par]).wait()
        pltpu.make_async_copy(item_emb_w.at[pl.ds(0, CHUNK_A), :],
                              irows_v.at[par], sem_i[par]).wait()

    def gather(par):
        pltpu.async_copy(cate_emb_w.at[cidx_v.at[par]], crows_v.at[par],
                         sem_g[par])

    def gather_wait(par):
        pltpu.make_async_copy(cate_emb_w.at[cidx_v.at[0]], crows_v.at[par],
                              sem_g[par]).wait()

    def pack_chunk(par):
        def prow(i, _):
            for u in range(8):
                r = i * 8 + u
                fused_v[par, r, :] = plsc.pack(irows_v[par, r, :],
                                               crows_v[par, r, :], format=_PK)
            return 0
        lax.fori_loop(0, CHUNK_A // 8, prow, 0)

    def out_copy(c, par):
        bc = pl.multiple_of(base + c * CHUNK_A, 8)
        pltpu.async_copy(fused_v.at[par], fb_out.at[pl.ds(bc, CHUNK_A)],
                         sem_o[par])

    def out_wait(par):
        pltpu.make_async_copy(fused_v.at[par], fb_out.at[pl.ds(0, CHUNK_A)],
                              sem_o[par]).wait()

    in_copy(0, 0)
    in_copy(1, 1)
    in_wait(0)
    gather(0)
    for c in range(NCH):
        par = c % 2
        if c + 1 < NCH:
            in_wait(1 - par)
            gather(1 - par)
        gather_wait(par)
        if c >= 2:
            out_wait(par)
        pack_chunk(par)
        out_copy(c, par)
        if c + 2 < NCH:
            in_copy(c + 2, par)
    for par in range(min(2, NCH)):
        out_wait(par)


@functools.partial(
    pl.kernel,
    out_type=jax.ShapeDtypeStruct((B, OUTW), jnp.float32),
    mesh=_mesh,
    scratch_types=[
        pltpu.VMEM((2, GROUP_B, L), jnp.int32),
        pltpu.VMEM((4, CHUNK, 2 * EMB_HALF), jnp.bfloat16),
        pltpu.VMEM((B_PER_W, OUTW), jnp.float32),
        pltpu.VMEM((B_PER_W,), jnp.int32),
        pltpu.VMEM((B_PER_W, 2 * EMB_HALF), jnp.bfloat16),
        pltpu.VMEM((B_PER_W,), jnp.float32),
        pltpu.SemaphoreType.DMA,
        pltpu.SemaphoreType.DMA,
        pltpu.SemaphoreType.DMA,
        pltpu.SemaphoreType.DMA,
        pltpu.SemaphoreType.DMA,
        pltpu.SemaphoreType.DMA,
        pltpu.SemaphoreType.DMA,
    ],
    compiler_params=_sc_params,
)
def _hist_sums(fb, hist2d, item, bias_flat,
               us_all_hbm,
               idx_v, rows, us_st,
               item_idx_v, ie_rows_v, bias_v,
               sem_idx0, sem_idx1, sem_rows0, sem_rows1, sem_rows2,
               sem_rows3, sem_misc):
    wid = lax.axis_index("s") * NC + lax.axis_index("c")
    base_b = wid * B_PER_W

    pltpu.sync_copy(item.at[pl.ds(pl.multiple_of(base_b, 8), B_PER_W)],
                    item_idx_v)
    pltpu.async_copy(fb.at[item_idx_v], ie_rows_v, sem_misc)
    pltpu.async_copy(bias_flat.at[item_idx_v], bias_v, sem_misc)

    sem_idx = (sem_idx0, sem_idx1)
    sem_rows = (sem_rows0, sem_rows1, sem_rows2, sem_rows3)

    def idx_copy(g, par):
        gb = pl.multiple_of(base_b + g * GROUP_B, 8)
        pltpu.async_copy(hist2d.at[pl.ds(gb, GROUP_B), :], idx_v.at[par],
                         sem_idx[par])

    def idx_wait(par):
        pltpu.make_async_copy(hist2d.at[pl.ds(0, GROUP_B), :],
                              idx_v.at[par], sem_idx[par]).wait()

    def issue_gather(ipar, off, rp):
        for j in range(NB):
            sl = idx_v.at[ipar, off // L + j, :]
            pltpu.async_copy(fb.at[sl], rows.at[rp, pl.ds(j * L, L), :],
                             sem_rows[rp])

    def rows_wait(rp):
        for j in range(NB):
            pltpu.make_async_copy(fb.at[idx_v.at[0, 0, :]],
                                  rows.at[rp, pl.ds(j * L, L), :],
                                  sem_rows[rp]).wait()

    UNROLL = 40
    NSTEPS = L // UNROLL

    def acc_chunk(c, rp):
        def bbody(b4, _):
            zf = jnp.zeros((EMB_HALF,), jnp.float32)

            def lbody(m, accs):
                ai, ac = accs
                r0 = b4 * L + m * UNROLL
                for h in range(2):
                    zb = [jnp.zeros((2 * EMB_HALF,), jnp.bfloat16)
                          for _ in range(4)]
                    for l in range(UNROLL // 2):
                        r = r0 + h * (UNROLL // 2) + l
                        zb[l % 4] = zb[l % 4] + rows[rp, r, :]
                    zz = (zb[0] + zb[1]) + (zb[2] + zb[3])
                    pi, pc = plsc.unpack(zz, format=_PK)
                    ai = ai + pi
                    ac = ac + pc
                return ai, ac

            ui, uc = lax.fori_loop(0, NSTEPS, lbody, (zf, zf))
            b_local = c * NB + b4
            us_st[b_local, pl.ds(0, EMB_HALF)] = ui
            us_st[b_local, pl.ds(EMB_HALF, EMB_HALF)] = uc
            return 0

        lax.fori_loop(0, NB, bbody, 0)

    def group_body(g, gpar):
        for s in range(SUBS):
            c = g * SUBS + s
            rp = s
            if s < SUBS - 2:
                issue_gather(gpar, (s + 2) * CHUNK, s + 2)
            elif s == SUBS - 2:
                @pl.when(g + 1 < NGROUPS)
                def _():
                    idx_wait(1 - gpar)
                    issue_gather(1 - gpar, 0, 0)
            else:
                @pl.when(g + 1 < NGROUPS)
                def _():
                    issue_gather(1 - gpar, CHUNK, 1)
            rows_wait(rp)
            acc_chunk(c, rp)
        @pl.when(g + 2 < NGROUPS)
        def _():
            idx_copy(g + 2, gpar)

    idx_copy(0, 0)
    idx_copy(1, 1)
    idx_wait(0)
    issue_gather(0, 0, 0)
    issue_gather(0, CHUNK, 1)

    def macro(i, _):
        group_body(2 * i, 0)
        group_body(2 * i + 1, 1)
        return 0

    lax.fori_loop(0, NGROUPS // 2, macro, 0)

    pltpu.make_async_copy(fb.at[item_idx_v], ie_rows_v, sem_misc).wait()
    pltpu.make_async_copy(bias_flat.at[item_idx_v], bias_v, sem_misc).wait()

    def ie_row(i, _):
        for u in range(8):
            r = i * 8 + u
            pi, pc = plsc.unpack(ie_rows_v[r, :], format=_PK)
            us_st[r, pl.ds(2 * EMB_HALF, EMB_HALF)] = pi
            us_st[r, pl.ds(3 * EMB_HALF, EMB_HALF)] = pc
        return 0

    lax.fori_loop(0, B_PER_W // 8, ie_row, 0)

    lane = lax.iota(jnp.int32, 16)
    col = jnp.full((16,), 4 * EMB_HALF, jnp.int32)

    def bias_row(i, _):
        vals = bias_v[pl.ds(i * 16, 16)]
        plsc.store_scatter(us_st, [lane + i * 16, col], vals)
        return 0

    lax.fori_loop(0, B_PER_W // 16, bias_row, 0)

    ob = pl.ds(pl.multiple_of(base_b, 8), B_PER_W)
    pltpu.sync_copy(us_st, us_all_hbm.at[ob, :])


def _tail_body(xa, w1, b1r, w2, b2r, o):
    inv_l = 1.0 / L
    x = xa[...]
    ue = x[:, 0:2 * EMB_HALF] * inv_l
    ie = x[:, 2 * EMB_HALF:4 * EMB_HALF]
    bz = x[:, 4 * EMB_HALF:4 * EMB_HALF + 1]
    wide = jnp.sum(ue * ie, axis=1, keepdims=True)
    xx = jnp.concatenate([ue, ie], axis=1)
    h = jnp.maximum(
        jnp.dot(xx, w1[...], preferred_element_type=jnp.float32) + b1r[...],
        0.0)
    deep = jnp.dot(h, w2[...], preferred_element_type=jnp.float32) + b2r[...]
    o[...] = jax.nn.sigmoid(bz + wide + deep)[:, 0]


def _tail(us_all, W1, b1r, W2, b2r):
    BLK = 4096
    return pl.pallas_call(
        _tail_body,
        grid=(B // BLK,),
        in_specs=[pl.BlockSpec((BLK, OUTW), lambda i: (i, 0)),
                  pl.BlockSpec((2 * 2 * EMB_HALF, HID), lambda i: (0, 0)),
                  pl.BlockSpec((1, HID), lambda i: (0, 0)),
                  pl.BlockSpec((HID, 1), lambda i: (0, 0)),
                  pl.BlockSpec((1, 1), lambda i: (0, 0))],
        out_specs=pl.BlockSpec((BLK,), lambda i: (i,)),
        out_shape=jax.ShapeDtypeStruct((B,), jnp.float32),
    )(us_all, W1, b1r, W2, b2r)


def kernel(item_emb_w, cate_emb_w, item_bias, W1, b1, W2, b2,
           cate_list, user, item, hist_item, neg_hist_item):
    fb = _build_fb(item_emb_w, cate_emb_w, cate_list)
    us_all = _hist_sums(fb, hist_item, item, item_bias.reshape(-1))
    out = _tail(us_all, W1, b1.reshape(1, HID), W2, b2.reshape(1, 1))
    return out.reshape(B, 1)
